# Initial kernel scaffold; baseline (speedup 1.0000x reference)
import jax
import jax.numpy as jnp
from jax.experimental import pallas as pl


def kernel(x, pe, W_emb, b_emb, s_d, s_hb, s_lW, s_lb, s_fW, s_fb, f_d, f_hb, f_lW, f_lb, f_fW, f_fb, n_g, n_b, n_a, fn_g, fn_b, fn_a, gate_W, gate_b, edge_index, batch):
    raise NotImplementedError("write your pallas kernel here")


# speedup vs baseline: 3.1233x; 3.1233x over previous
import functools

import jax
import jax.numpy as jnp
from jax.experimental import pallas as pl
from jax.experimental.pallas import tpu as pltpu

_N = 10000
_C = 128
_G = 16
_K = 3
_P = 3
_L = 2
_SLOPE = 0.01
_EBLK = 2000


def _leaky(x):
    return jnp.where(x >= 0, x, _SLOPE * x)


def _embed_body(x_ref, w_ref, b_ref, sw_ref, sb_ref, h_ref, xn_ref, xt_ref):
    x = x_ref[...]
    h = jnp.dot(x, w_ref[...], preferred_element_type=jnp.float32) + b_ref[0:1, :]
    h_ref[...] = h
    nrm = jnp.sqrt(jnp.sum(h * h, axis=1, keepdims=True)) + 1e-8
    xn_ref[...] = h / nrm
    xt_ref[...] = jnp.dot(h, sw_ref[...], preferred_element_type=jnp.float32) + sb_ref[0:1, :]


def _embed(x, w, b8, sw, sb8):
    t = 2000
    grid = _N // t
    return pl.pallas_call(
        _embed_body,
        grid=(grid,),
        in_specs=[
            pl.BlockSpec((t, _C), lambda i: (i, 0)),
            pl.BlockSpec((_C, _C), lambda i: (0, 0)),
            pl.BlockSpec((8, _C), lambda i: (0, 0)),
            pl.BlockSpec((_C, _C), lambda i: (0, 0)),
            pl.BlockSpec((8, _C), lambda i: (0, 0)),
        ],
        out_specs=[
            pl.BlockSpec((t, _C), lambda i: (i, 0)),
            pl.BlockSpec((t, _C), lambda i: (i, 0)),
            pl.BlockSpec((t, _C), lambda i: (i, 0)),
        ],
        out_shape=[jax.ShapeDtypeStruct((_N, _C), jnp.float32)] * 3,
    )(x, w, b8, sw, sb8)


def _knn_body(xc_ref, xa_ref, brow_ref, bcol_ref, out_ref):
    xc = xc_ref[...]
    xa = xa_ref[...]
    t = xc.shape[0]
    sim = jax.lax.dot_general(xc, xa, (((1,), (1,)), ((), ())),
                              preferred_element_type=jnp.float32)
    brow = brow_ref[...][:, 0:1]
    bcol = bcol_ref[0:1, :]
    sim = jnp.where(brow != bcol, 2.0, sim)
    rid = jax.lax.broadcasted_iota(jnp.int32, (t, _N), 0) + pl.program_id(0) * t
    cid = jax.lax.broadcasted_iota(jnp.int32, (t, _N), 1)
    sim = jnp.where(rid == cid, 2.0, sim)
    idxs = []
    for _ in range(_K):
        m = jnp.min(sim, axis=1, keepdims=True)
        idx = jnp.min(jnp.where(sim == m, cid, _N), axis=1, keepdims=True)
        idxs.append(idx)
        sim = jnp.where(cid == idx, 4.0, sim)
    lane = jax.lax.broadcasted_iota(jnp.int32, (t, _C), 1)
    out = jnp.zeros((t, _C), jnp.int32)
    for j in range(_K):
        out = jnp.where(lane == j, idxs[j], out)
    out_ref[...] = out


def _knn(xn, batchf, batchrow):
    t = 400
    grid = _N // t
    return pl.pallas_call(
        _knn_body,
        grid=(grid,),
        in_specs=[
            pl.BlockSpec((t, _C), lambda i: (i, 0)),
            pl.BlockSpec((_N, _C), lambda i: (0, 0)),
            pl.BlockSpec((t, _C), lambda i: (i, 0)),
            pl.BlockSpec((8, _N), lambda i: (0, 0)),
        ],
        out_specs=pl.BlockSpec((t, _C), lambda i: (i, 0)),
        out_shape=jax.ShapeDtypeStruct((_N, _C), jnp.int32),
    )(xn, xn, batchf, batchrow)


def _edge_body(rows_ref, cols_ref, xt_ref, pe_ref, acc_ref):
    @pl.when(pl.program_id(0) == 0)
    def _():
        acc_ref[...] = jnp.zeros_like(acc_ref)

    ones = jnp.ones((1, _C), jnp.float32)

    def body(e, carry):
        r = rows_ref[0, 0, e]
        c = cols_ref[0, 0, e]
        xv = xt_ref[pl.ds(c, 1), :]
        d = pe_ref[pl.ds(c, 1), :] - pe_ref[pl.ds(r, 1), :]
        up = jnp.maximum(d, 0.0)
        dn = jnp.maximum(-d, 0.0)
        row = jnp.concatenate([xv, up * xv, dn * xv, ones], axis=1)
        acc_ref[pl.ds(r, 1), :] = acc_ref[pl.ds(r, 1), :] + row
        return carry

    jax.lax.fori_loop(0, _EBLK, body, 0)


def _edge(rows3, cols3, xt, pe128):
    nblk = rows3.shape[0]
    return pl.pallas_call(
        _edge_body,
        grid=(nblk,),
        in_specs=[
            pl.BlockSpec((1, 1, _EBLK), lambda i: (i, 0, 0), memory_space=pltpu.SMEM),
            pl.BlockSpec((1, 1, _EBLK), lambda i: (i, 0, 0), memory_space=pltpu.SMEM),
            pl.BlockSpec((_N, _C), lambda i: (0, 0)),
            pl.BlockSpec((_N, _C), lambda i: (0, 0)),
        ],
        out_specs=pl.BlockSpec((_N, 4 * _C), lambda i: (0, 0)),
        out_shape=jax.ShapeDtypeStruct((_N, 4 * _C), jnp.float32),
    )(rows3, cols3, xt, pe128)


def _fusion_math(xt, acc_ref, f0, f1, fu, fd, fb, d, hb):
    sp = acc_ref[:, 0:_C]
    su = acc_ref[:, _C:2 * _C]
    sd = acc_ref[:, 2 * _C:3 * _C]
    deg = acc_ref[:, 3 * _C:3 * _C + 1]
    avg = sp * (1.0 / deg)
    dv = d[0:_P, :]
    dm = jnp.max(dv, axis=0, keepdims=True)
    de = jnp.exp(dv - dm)
    dw = de / jnp.sum(de, axis=0, keepdims=True)
    out = jnp.zeros_like(xt)
    for p in range(_P):
        t = (jnp.dot(xt, f0[p], preferred_element_type=jnp.float32)
             + jnp.dot(avg, f1[p], preferred_element_type=jnp.float32)
             + (p + 1.0) * (jnp.dot(su, fu[p], preferred_element_type=jnp.float32)
                            + jnp.dot(sd, fd[p], preferred_element_type=jnp.float32))
             + fb[p:p + 1, :])
        out = out + _leaky(t) * dw[p:p + 1, :] + hb[p:p + 1, :]
    return out


def _seg_sum(oht, x):
    return jax.lax.dot_general(oht, x, (((1,), (0,)), ((), ())),
                               preferred_element_type=jnp.float32)


def _seg_bcast(oht, v):
    return jax.lax.dot_general(oht, v, (((0,), (0,)), ((), ())),
                               preferred_element_type=jnp.float32)


def _gnorm_math(x, oht, cntg, g, b, a):
    mean = _seg_sum(oht, x) / cntg
    xc = x - a[0:1, :] * _seg_bcast(oht, mean)
    var = _seg_sum(oht, xc * xc) / cntg
    varb = _seg_bcast(oht, var)
    return g[0:1, :] * xc / jnp.sqrt(varb + 1e-5) + b[0:1, :]


def _make_cnt(oht):
    cnt = jnp.sum(oht, axis=1, keepdims=True)
    return jnp.maximum(jnp.broadcast_to(cnt, (_G, _C)), 1.0)


def _fuse_b_body(xt_ref, acc_ref, f0_ref, f1_ref, fu_ref, fd_ref, fb_ref,
                 d_ref, hb_ref, g_ref, b_ref, a_ref, oht_ref,
                 nw_ref, nb_ref, h1_ref, xtn_ref):
    xt = xt_ref[...]
    out = _fusion_math(xt, acc_ref, f0_ref[...], f1_ref[...], fu_ref[...],
                       fd_ref[...], fb_ref[...], d_ref[...], hb_ref[...])
    oht = oht_ref[...]
    cntg = _make_cnt(oht)
    h1 = _leaky(_gnorm_math(out, oht, cntg, g_ref[...], b_ref[...], a_ref[...]))
    h1_ref[...] = h1
    xtn_ref[...] = jnp.dot(h1, nw_ref[...], preferred_element_type=jnp.float32) + nb_ref[0:1, :]


def _fuse_n_body(xt_ref, acc_ref, f0_ref, f1_ref, fu_ref, fd_ref, fb_ref,
                 d_ref, hb_ref, g_ref, b_ref, a_ref, oht_ref, h2_ref):
    xt = xt_ref[...]
    out = _fusion_math(xt, acc_ref, f0_ref[...], f1_ref[...], fu_ref[...],
                       fd_ref[...], fb_ref[...], d_ref[...], hb_ref[...])
    oht = oht_ref[...]
    cntg = _make_cnt(oht)
    h2_ref[...] = _leaky(_gnorm_math(out, oht, cntg, g_ref[...], b_ref[...], a_ref[...]))


def _gate_mid_body(h1_ref, h2_ref, prev_ref, gw1_ref, gw2_ref, gb_ref,
                   nw_ref, nb_ref, hn_ref, xtn_ref):
    h1 = h1_ref[...]
    h2 = h2_ref[...]
    gate = jax.nn.sigmoid(jnp.dot(h1, gw1_ref[...], preferred_element_type=jnp.float32)
                          + jnp.dot(h2, gw2_ref[...], preferred_element_type=jnp.float32)
                          + gb_ref[0:1, :])
    hn = gate * h1 + (1.0 - gate) * h2 + prev_ref[...]
    hn_ref[...] = hn
    xtn_ref[...] = jnp.dot(hn, nw_ref[...], preferred_element_type=jnp.float32) + nb_ref[0:1, :]


def _gate_last_body(h1_ref, h2_ref, prev_ref, gw1_ref, gw2_ref, gb_ref,
                    oht_ref, gf_ref):
    h1 = h1_ref[...]
    h2 = h2_ref[...]
    gate = jax.nn.sigmoid(jnp.dot(h1, gw1_ref[...], preferred_element_type=jnp.float32)
                          + jnp.dot(h2, gw2_ref[...], preferred_element_type=jnp.float32)
                          + gb_ref[0:1, :])
    hn = gate * h1 + (1.0 - gate) * h2 + prev_ref[...]
    oht = oht_ref[...]
    gf_ref[...] = _seg_sum(oht, hn) / _make_cnt(oht)


_FULL = lambda shape: pl.BlockSpec(shape, lambda: tuple(0 for _ in shape))


def _fuse_b(xt, acc, f0, f1, fu, fd, fb8, d8, hb8, g8, b8, a8, oht, nw, nb8):
    args = (xt, acc, f0, f1, fu, fd, fb8, d8, hb8, g8, b8, a8, oht, nw, nb8)
    return pl.pallas_call(
        _fuse_b_body,
        in_specs=[_FULL(a.shape) for a in args],
        out_specs=[_FULL((_N, _C)), _FULL((_N, _C))],
        out_shape=[jax.ShapeDtypeStruct((_N, _C), jnp.float32)] * 2,
    )(*args)


def _fuse_n(xt, acc, f0, f1, fu, fd, fb8, d8, hb8, g8, b8, a8, oht):
    args = (xt, acc, f0, f1, fu, fd, fb8, d8, hb8, g8, b8, a8, oht)
    return pl.pallas_call(
        _fuse_n_body,
        in_specs=[_FULL(a.shape) for a in args],
        out_specs=_FULL((_N, _C)),
        out_shape=jax.ShapeDtypeStruct((_N, _C), jnp.float32),
    )(*args)


def _gate_mid(h1, h2, prev, gw1, gw2, gb8, nw, nb8):
    args = (h1, h2, prev, gw1, gw2, gb8, nw, nb8)
    return pl.pallas_call(
        _gate_mid_body,
        in_specs=[_FULL(a.shape) for a in args],
        out_specs=[_FULL((_N, _C)), _FULL((_N, _C))],
        out_shape=[jax.ShapeDtypeStruct((_N, _C), jnp.float32)] * 2,
    )(*args)


def _gate_last(h1, h2, prev, gw1, gw2, gb8, oht):
    args = (h1, h2, prev, gw1, gw2, gb8, oht)
    return pl.pallas_call(
        _gate_last_body,
        in_specs=[_FULL(a.shape) for a in args],
        out_specs=_FULL((_G, _C)),
        out_shape=jax.ShapeDtypeStruct((_G, _C), jnp.float32),
    )(*args)


def _pad8(a):
    return jnp.concatenate([a, jnp.zeros((8 - a.shape[0],) + a.shape[1:], a.dtype)], 0)


def _row8(v):
    return jnp.broadcast_to(v[None, :], (8, v.shape[0])).astype(jnp.float32)


def kernel(x, pe, W_emb, b_emb, s_d, s_hb, s_lW, s_lb, s_fW, s_fb,
           f_d, f_hb, f_lW, f_lb, f_fW, f_fb, n_g, n_b, n_a,
           fn_g, fn_b, fn_a, gate_W, gate_b, edge_index, batch):
    ar = jnp.arange(_N, dtype=jnp.int32)
    batchf = jnp.broadcast_to(batch[:, None], (_N, _C)).astype(jnp.float32)
    batchrow = _row8(batch)
    oht = (batch[None, :] == jnp.arange(_G, dtype=batch.dtype)[:, None]).astype(jnp.float32)
    pe128 = jnp.broadcast_to(pe[:, 0:1], (_N, _C))

    sf0 = s_fW[:, :, 0:_C, :]
    sf1 = s_fW[:, :, _C:2 * _C, :]
    sfu = s_fW[:, :, 2 * _C::2, :]
    sfd = s_fW[:, :, 2 * _C + 1::2, :]
    ff0 = f_fW[:, :, 0:_C, :]
    ff1 = f_fW[:, :, _C:2 * _C, :]
    ffu = f_fW[:, :, 2 * _C::2, :]
    ffd = f_fW[:, :, 2 * _C + 1::2, :]
    gw1 = gate_W[0:_C, :]
    gw2 = gate_W[_C:2 * _C, :]
    gb8 = _row8(gate_b)
    bemb8 = _row8(b_emb)

    h, xn, xt = _embed(x, W_emb, bemb8, s_lW[0], _row8(s_lb[0]))
    knn = _knn(xn, batchf, batchrow)

    sm_rows = jnp.concatenate([edge_index[0].astype(jnp.int32), ar])
    sm_cols = jnp.concatenate([edge_index[1].astype(jnp.int32), ar])
    fm_rows = jnp.concatenate([jnp.repeat(ar, _K), ar, ar])
    fm_cols = jnp.concatenate([knn[:, 0:_K].reshape(-1), ar, ar])
    sm_rows3 = sm_rows.reshape(-1, 1, _EBLK)
    sm_cols3 = sm_cols.reshape(-1, 1, _EBLK)
    fm_rows3 = fm_rows.reshape(-1, 1, _EBLK)
    fm_cols3 = fm_cols.reshape(-1, 1, _EBLK)

    h_cur = h
    for i in range(_L):
        acc_s = _edge(sm_rows3, sm_cols3, xt, pe128)
        h1, xtf = _fuse_b(xt, acc_s, sf0[i], sf1[i], sfu[i], sfd[i],
                          _pad8(s_fb[i]), _pad8(s_d[i]), _pad8(s_hb[i]),
                          _row8(n_g[i]), _row8(n_b[i]), _row8(n_a[i]),
                          oht, f_lW[i], _row8(f_lb[i]))
        acc_f = _edge(fm_rows3, fm_cols3, xtf, pe128)
        h2 = _fuse_n(xtf, acc_f, ff0[i], ff1[i], ffu[i], ffd[i],
                     _pad8(f_fb[i]), _pad8(f_d[i]), _pad8(f_hb[i]),
                     _row8(fn_g[i]), _row8(fn_b[i]), _row8(fn_a[i]), oht)
        if i < _L - 1:
            h_cur, xt = _gate_mid(h1, h2, h_cur, gw1, gw2, gb8,
                                  s_lW[i + 1], _row8(s_lb[i + 1]))
        else:
            h_cur = _gate_last(h1, h2, h_cur, gw1, gw2, gb8, oht)
    return h_cur


# speedup vs baseline: 4.7348x; 1.5160x over previous
import functools

import jax
import jax.numpy as jnp
from jax.experimental import pallas as pl
from jax.experimental.pallas import tpu as pltpu

_N = 10000
_C = 128
_G = 16
_K = 3
_P = 3
_L = 2
_SLOPE = 0.01
_EBLK = 2000


def _leaky(x):
    return jnp.where(x >= 0, x, _SLOPE * x)


def _embed_body(x_ref, w_ref, b_ref, sw_ref, sb_ref, h_ref, xn_ref, xt_ref):
    x = x_ref[...]
    h = jnp.dot(x, w_ref[...], preferred_element_type=jnp.float32) + b_ref[0:1, :]
    h_ref[...] = h
    nrm = jnp.sqrt(jnp.sum(h * h, axis=1, keepdims=True)) + 1e-8
    xn_ref[...] = h / nrm
    xt_ref[...] = jnp.dot(h, sw_ref[...], preferred_element_type=jnp.float32) + sb_ref[0:1, :]


def _embed(x, w, b8, sw, sb8):
    t = 2000
    grid = _N // t
    return pl.pallas_call(
        _embed_body,
        grid=(grid,),
        in_specs=[
            pl.BlockSpec((t, _C), lambda i: (i, 0)),
            pl.BlockSpec((_C, _C), lambda i: (0, 0)),
            pl.BlockSpec((8, _C), lambda i: (0, 0)),
            pl.BlockSpec((_C, _C), lambda i: (0, 0)),
            pl.BlockSpec((8, _C), lambda i: (0, 0)),
        ],
        out_specs=[
            pl.BlockSpec((t, _C), lambda i: (i, 0)),
            pl.BlockSpec((t, _C), lambda i: (i, 0)),
            pl.BlockSpec((t, _C), lambda i: (i, 0)),
        ],
        out_shape=[jax.ShapeDtypeStruct((_N, _C), jnp.float32)] * 3,
    )(x, w, b8, sw, sb8)


def _knn_body(xc_ref, xa_ref, brow_ref, bcol_ref, out_ref):
    xc = xc_ref[...]
    xa = xa_ref[...]
    t = xc.shape[0]
    sim = jax.lax.dot_general(xc, xa, (((1,), (1,)), ((), ())),
                              preferred_element_type=jnp.float32)
    brow = brow_ref[...][:, 0:1]
    bcol = bcol_ref[0:1, :]
    sim = jnp.where(brow != bcol, 2.0, sim)
    rid = jax.lax.broadcasted_iota(jnp.int32, (t, _N), 0) + pl.program_id(0) * t
    cid = jax.lax.broadcasted_iota(jnp.int32, (t, _N), 1)
    sim = jnp.where(rid == cid, 2.0, sim)
    idxs = []
    for _ in range(_K):
        m = jnp.min(sim, axis=1, keepdims=True)
        idx = jnp.min(jnp.where(sim == m, cid, _N), axis=1, keepdims=True)
        idxs.append(idx)
        sim = jnp.where(cid == idx, 4.0, sim)
    lane = jax.lax.broadcasted_iota(jnp.int32, (t, _C), 1)
    out = jnp.zeros((t, _C), jnp.int32)
    for j in range(_K):
        out = jnp.where(lane == j, idxs[j], out)
    out_ref[...] = out


def _knn(xn, batchf, batchrow):
    t = 400
    grid = _N // t
    return pl.pallas_call(
        _knn_body,
        grid=(grid,),
        in_specs=[
            pl.BlockSpec((t, _C), lambda i: (i, 0)),
            pl.BlockSpec((_N, _C), lambda i: (0, 0)),
            pl.BlockSpec((t, _C), lambda i: (i, 0)),
            pl.BlockSpec((8, _N), lambda i: (0, 0)),
        ],
        out_specs=pl.BlockSpec((t, _C), lambda i: (i, 0)),
        out_shape=jax.ShapeDtypeStruct((_N, _C), jnp.int32),
    )(xn, xn, batchf, batchrow)


def _edge_body(rows_ref, cols_ref, xt_ref, pe_ref, acc0_ref, acc1_ref):
    @pl.when(pl.program_id(0) == 0)
    def _():
        acc0_ref[...] = jnp.zeros_like(acc0_ref)
        acc1_ref[...] = jnp.zeros_like(acc1_ref)

    ones = jnp.ones((1, _C), jnp.float32)

    def one(e, acc_ref):
        r = rows_ref[0, 0, e]
        c = cols_ref[0, 0, e]
        xv = xt_ref[pl.ds(c, 1), :]
        d = pe_ref[pl.ds(c, 1), :] - pe_ref[pl.ds(r, 1), :]
        up = jnp.maximum(d, 0.0)
        dn = jnp.maximum(-d, 0.0)
        row = jnp.concatenate([xv, up * xv, dn * xv, ones], axis=1)
        acc_ref[pl.ds(r, 1), :] = acc_ref[pl.ds(r, 1), :] + row

    def body(k, carry):
        one(2 * k, acc0_ref)
        one(2 * k + 1, acc1_ref)
        return carry

    jax.lax.fori_loop(0, _EBLK // 2, body, 0)


def _accsum_body(a_ref, b_ref, o_ref):
    o_ref[...] = a_ref[...] + b_ref[...]


def _edge(rows3, cols3, xt, pe128):
    nblk = rows3.shape[0]
    acc0, acc1 = pl.pallas_call(
        _edge_body,
        grid=(nblk,),
        in_specs=[
            pl.BlockSpec((1, 1, _EBLK), lambda i: (i, 0, 0), memory_space=pltpu.SMEM),
            pl.BlockSpec((1, 1, _EBLK), lambda i: (i, 0, 0), memory_space=pltpu.SMEM),
            pl.BlockSpec((_N, _C), lambda i: (0, 0)),
            pl.BlockSpec((_N, _C), lambda i: (0, 0)),
        ],
        out_specs=[pl.BlockSpec((_N, 4 * _C), lambda i: (0, 0))] * 2,
        out_shape=[jax.ShapeDtypeStruct((_N, 4 * _C), jnp.float32)] * 2,
    )(rows3, cols3, xt, pe128)
    t = 2000
    return pl.pallas_call(
        _accsum_body,
        grid=(_N // t,),
        in_specs=[pl.BlockSpec((t, 4 * _C), lambda i: (i, 0))] * 2,
        out_specs=pl.BlockSpec((t, 4 * _C), lambda i: (i, 0)),
        out_shape=jax.ShapeDtypeStruct((_N, 4 * _C), jnp.float32),
    )(acc0, acc1)


def _fusion_math(xt, acc_ref, f0, f1, fu, fd, fb, d, hb):
    sp = acc_ref[:, 0:_C]
    su = acc_ref[:, _C:2 * _C]
    sd = acc_ref[:, 2 * _C:3 * _C]
    deg = acc_ref[:, 3 * _C:3 * _C + 1]
    avg = sp * (1.0 / deg)
    dv = d[0:_P, :]
    dm = jnp.max(dv, axis=0, keepdims=True)
    de = jnp.exp(dv - dm)
    dw = de / jnp.sum(de, axis=0, keepdims=True)
    out = jnp.zeros_like(xt)
    for p in range(_P):
        t = (jnp.dot(xt, f0[p], preferred_element_type=jnp.float32)
             + jnp.dot(avg, f1[p], preferred_element_type=jnp.float32)
             + (p + 1.0) * (jnp.dot(su, fu[p], preferred_element_type=jnp.float32)
                            + jnp.dot(sd, fd[p], preferred_element_type=jnp.float32))
             + fb[p:p + 1, :])
        out = out + _leaky(t) * dw[p:p + 1, :] + hb[p:p + 1, :]
    return out


def _seg_sum(oht, x):
    return jax.lax.dot_general(oht, x, (((1,), (0,)), ((), ())),
                               preferred_element_type=jnp.float32)


def _seg_bcast(oht, v):
    return jax.lax.dot_general(oht, v, (((0,), (0,)), ((), ())),
                               preferred_element_type=jnp.float32)


def _gnorm_math(x, oht, cntg, g, b, a):
    mean = _seg_sum(oht, x) / cntg
    xc = x - a[0:1, :] * _seg_bcast(oht, mean)
    var = _seg_sum(oht, xc * xc) / cntg
    varb = _seg_bcast(oht, var)
    return g[0:1, :] * xc / jnp.sqrt(varb + 1e-5) + b[0:1, :]


def _make_cnt(oht):
    cnt = jnp.sum(oht, axis=1, keepdims=True)
    return jnp.maximum(jnp.broadcast_to(cnt, (_G, _C)), 1.0)


def _fuse_b_body(xt_ref, acc_ref, f0_ref, f1_ref, fu_ref, fd_ref, fb_ref,
                 d_ref, hb_ref, g_ref, b_ref, a_ref, oht_ref,
                 nw_ref, nb_ref, h1_ref, xtn_ref):
    xt = xt_ref[...]
    out = _fusion_math(xt, acc_ref, f0_ref[...], f1_ref[...], fu_ref[...],
                       fd_ref[...], fb_ref[...], d_ref[...], hb_ref[...])
    oht = oht_ref[...]
    cntg = _make_cnt(oht)
    h1 = _leaky(_gnorm_math(out, oht, cntg, g_ref[...], b_ref[...], a_ref[...]))
    h1_ref[...] = h1
    xtn_ref[...] = jnp.dot(h1, nw_ref[...], preferred_element_type=jnp.float32) + nb_ref[0:1, :]


def _fuse_n_body(xt_ref, acc_ref, f0_ref, f1_ref, fu_ref, fd_ref, fb_ref,
                 d_ref, hb_ref, g_ref, b_ref, a_ref, oht_ref, h2_ref):
    xt = xt_ref[...]
    out = _fusion_math(xt, acc_ref, f0_ref[...], f1_ref[...], fu_ref[...],
                       fd_ref[...], fb_ref[...], d_ref[...], hb_ref[...])
    oht = oht_ref[...]
    cntg = _make_cnt(oht)
    h2_ref[...] = _leaky(_gnorm_math(out, oht, cntg, g_ref[...], b_ref[...], a_ref[...]))


def _gate_mid_body(h1_ref, h2_ref, prev_ref, gw1_ref, gw2_ref, gb_ref,
                   nw_ref, nb_ref, hn_ref, xtn_ref):
    h1 = h1_ref[...]
    h2 = h2_ref[...]
    gate = jax.nn.sigmoid(jnp.dot(h1, gw1_ref[...], preferred_element_type=jnp.float32)
                          + jnp.dot(h2, gw2_ref[...], preferred_element_type=jnp.float32)
                          + gb_ref[0:1, :])
    hn = gate * h1 + (1.0 - gate) * h2 + prev_ref[...]
    hn_ref[...] = hn
    xtn_ref[...] = jnp.dot(hn, nw_ref[...], preferred_element_type=jnp.float32) + nb_ref[0:1, :]


def _gate_last_body(h1_ref, h2_ref, prev_ref, gw1_ref, gw2_ref, gb_ref,
                    oht_ref, gf_ref):
    h1 = h1_ref[...]
    h2 = h2_ref[...]
    gate = jax.nn.sigmoid(jnp.dot(h1, gw1_ref[...], preferred_element_type=jnp.float32)
                          + jnp.dot(h2, gw2_ref[...], preferred_element_type=jnp.float32)
                          + gb_ref[0:1, :])
    hn = gate * h1 + (1.0 - gate) * h2 + prev_ref[...]
    oht = oht_ref[...]
    gf_ref[...] = _seg_sum(oht, hn) / _make_cnt(oht)


_FULL = lambda shape: pl.BlockSpec(shape, lambda: tuple(0 for _ in shape))


def _fuse_b(xt, acc, f0, f1, fu, fd, fb8, d8, hb8, g8, b8, a8, oht, nw, nb8):
    args = (xt, acc, f0, f1, fu, fd, fb8, d8, hb8, g8, b8, a8, oht, nw, nb8)
    return pl.pallas_call(
        _fuse_b_body,
        in_specs=[_FULL(a.shape) for a in args],
        out_specs=[_FULL((_N, _C)), _FULL((_N, _C))],
        out_shape=[jax.ShapeDtypeStruct((_N, _C), jnp.float32)] * 2,
    )(*args)


def _fuse_n(xt, acc, f0, f1, fu, fd, fb8, d8, hb8, g8, b8, a8, oht):
    args = (xt, acc, f0, f1, fu, fd, fb8, d8, hb8, g8, b8, a8, oht)
    return pl.pallas_call(
        _fuse_n_body,
        in_specs=[_FULL(a.shape) for a in args],
        out_specs=_FULL((_N, _C)),
        out_shape=jax.ShapeDtypeStruct((_N, _C), jnp.float32),
    )(*args)


def _gate_mid(h1, h2, prev, gw1, gw2, gb8, nw, nb8):
    args = (h1, h2, prev, gw1, gw2, gb8, nw, nb8)
    return pl.pallas_call(
        _gate_mid_body,
        in_specs=[_FULL(a.shape) for a in args],
        out_specs=[_FULL((_N, _C)), _FULL((_N, _C))],
        out_shape=[jax.ShapeDtypeStruct((_N, _C), jnp.float32)] * 2,
    )(*args)


def _gate_last(h1, h2, prev, gw1, gw2, gb8, oht):
    args = (h1, h2, prev, gw1, gw2, gb8, oht)
    return pl.pallas_call(
        _gate_last_body,
        in_specs=[_FULL(a.shape) for a in args],
        out_specs=_FULL((_G, _C)),
        out_shape=jax.ShapeDtypeStruct((_G, _C), jnp.float32),
    )(*args)


def _pad8(a):
    return jnp.concatenate([a, jnp.zeros((8 - a.shape[0],) + a.shape[1:], a.dtype)], 0)


def _row8(v):
    return jnp.broadcast_to(v[None, :], (8, v.shape[0])).astype(jnp.float32)


def kernel(x, pe, W_emb, b_emb, s_d, s_hb, s_lW, s_lb, s_fW, s_fb,
           f_d, f_hb, f_lW, f_lb, f_fW, f_fb, n_g, n_b, n_a,
           fn_g, fn_b, fn_a, gate_W, gate_b, edge_index, batch):
    ar = jnp.arange(_N, dtype=jnp.int32)
    batchf = jnp.broadcast_to(batch[:, None], (_N, _C)).astype(jnp.float32)
    batchrow = _row8(batch)
    oht = (batch[None, :] == jnp.arange(_G, dtype=batch.dtype)[:, None]).astype(jnp.float32)
    pe128 = jnp.broadcast_to(pe[:, 0:1], (_N, _C))

    sf0 = s_fW[:, :, 0:_C, :]
    sf1 = s_fW[:, :, _C:2 * _C, :]
    sfu = s_fW[:, :, 2 * _C::2, :]
    sfd = s_fW[:, :, 2 * _C + 1::2, :]
    ff0 = f_fW[:, :, 0:_C, :]
    ff1 = f_fW[:, :, _C:2 * _C, :]
    ffu = f_fW[:, :, 2 * _C::2, :]
    ffd = f_fW[:, :, 2 * _C + 1::2, :]
    gw1 = gate_W[0:_C, :]
    gw2 = gate_W[_C:2 * _C, :]
    gb8 = _row8(gate_b)
    bemb8 = _row8(b_emb)

    h, xn, xt = _embed(x, W_emb, bemb8, s_lW[0], _row8(s_lb[0]))
    knn = _knn(xn, batchf, batchrow)

    sm_rows = jnp.concatenate([edge_index[0].astype(jnp.int32), ar])
    sm_cols = jnp.concatenate([edge_index[1].astype(jnp.int32), ar])
    fm_rows = jnp.concatenate([jnp.repeat(ar, _K), ar, ar])
    fm_cols = jnp.concatenate([knn[:, 0:_K].reshape(-1), ar, ar])
    sm_rows3 = sm_rows.reshape(-1, 1, _EBLK)
    sm_cols3 = sm_cols.reshape(-1, 1, _EBLK)
    fm_rows3 = fm_rows.reshape(-1, 1, _EBLK)
    fm_cols3 = fm_cols.reshape(-1, 1, _EBLK)

    h_cur = h
    for i in range(_L):
        acc_s = _edge(sm_rows3, sm_cols3, xt, pe128)
        h1, xtf = _fuse_b(xt, acc_s, sf0[i], sf1[i], sfu[i], sfd[i],
                          _pad8(s_fb[i]), _pad8(s_d[i]), _pad8(s_hb[i]),
                          _row8(n_g[i]), _row8(n_b[i]), _row8(n_a[i]),
                          oht, f_lW[i], _row8(f_lb[i]))
        acc_f = _edge(fm_rows3, fm_cols3, xtf, pe128)
        h2 = _fuse_n(xtf, acc_f, ff0[i], ff1[i], ffu[i], ffd[i],
                     _pad8(f_fb[i]), _pad8(f_d[i]), _pad8(f_hb[i]),
                     _row8(fn_g[i]), _row8(fn_b[i]), _row8(fn_a[i]), oht)
        if i < _L - 1:
            h_cur, xt = _gate_mid(h1, h2, h_cur, gw1, gw2, gb8,
                                  s_lW[i + 1], _row8(s_lb[i + 1]))
        else:
            h_cur = _gate_last(h1, h2, h_cur, gw1, gw2, gb8, oht)
    return h_cur
